# baseline (speedup 1.0000x reference)
import functools

import jax
import jax.numpy as jnp
from jax import lax
from jax.experimental import pallas as pl
from jax.experimental.pallas import tpu as pltpu
from jax.experimental.pallas import tpu_sc as plsc

_B, _C, _N = 8, 128, 65536
_K = 2048
_H1, _H2 = 64, 64
_OUT = 144
_NT = 4096
_L = 16
_NBINS = 1024
_CAND = 4096
_PAD_KEY = 0x3FFFFFFF


def _imp_body(h_ref, W1_ref, b1_ref, W2_ref, b2_ref, Wa_ref, ba_ref, out_ref):
    hb = h_ref[0]
    f1 = jax.lax.dot(W1_ref[...], hb, preferred_element_type=jnp.float32)
    f1 = f1 + b1_ref[...].T
    f1 = jnp.where(f1 >= 0, f1, 0.1 * f1)
    f2 = jax.lax.dot(W2_ref[...], f1, preferred_element_type=jnp.float32)
    f2 = f2 + b2_ref[...].T
    f2 = jnp.where(f2 >= 0, f2, 0.1 * f2)
    a = jax.lax.dot(Wa_ref[...], f2, preferred_element_type=jnp.float32)
    a = a + ba_ref[...]
    out_ref[...] = jnp.clip(jax.nn.sigmoid(a), 0.0, 1.0)[None]


def _importance(h, W1, b1, W2, b2, Wa, ba):
    grid = (_B, _N // _NT)
    return pl.pallas_call(
        _imp_body,
        grid=grid,
        in_specs=[
            pl.BlockSpec((1, _C, _NT), lambda b, n: (b, 0, n)),
            pl.BlockSpec((_H1, _C), lambda b, n: (0, 0)),
            pl.BlockSpec((1, _H1), lambda b, n: (0, 0)),
            pl.BlockSpec((_H2, _H1), lambda b, n: (0, 0)),
            pl.BlockSpec((1, _H2), lambda b, n: (0, 0)),
            pl.BlockSpec((1, _H2), lambda b, n: (0, 0)),
            pl.BlockSpec((1, 1), lambda b, n: (0, 0)),
        ],
        out_specs=pl.BlockSpec((1, 1, _NT), lambda b, n: (b, 0, n)),
        out_shape=jax.ShapeDtypeStruct((_B, 1, _N), jnp.float32),
    )(h, W1, b1.reshape(1, _H1), W2, b2.reshape(1, _H2), Wa,
      ba.reshape(1, 1)).reshape(_B, _N)


_U = 4


def _sc_topk_body(imp_hbm, inds_hbm, row_v, histp, histp2, hist_v, cka,
                  cia, ckb, cib):
    cid = lax.axis_index("c")
    sid = lax.axis_index("s")
    row = cid * (_B // 2) + sid

    @pl.when(sid < _B // 2)
    def _():
        pltpu.sync_copy(imp_hbm.at[row], row_v)
        zeros = jnp.zeros((_L,), jnp.int32)

        def zero_partials(g, _):
            for u in range(_U):
                histp[pl.ds((g * _U + u) * _L, _L)] = zeros
            return 0

        def hist_step(g, _):
            for u in range(_U):
                i = g * _U + u
                bits = lax.bitcast_convert_type(row_v[pl.ds(i * _L, _L)],
                                                jnp.int32)
                b = jnp.minimum(bits >> 20, _NBINS - 1)
                cnt, last = plsc.scan_count(b)
                plsc.addupdate_scatter(histp, [u * _NBINS + b], cnt,
                                       mask=last)
            return 0

        lax.fori_loop(0, _U * _NBINS // _L // _U, zero_partials, 0)
        lax.fori_loop(0, _N // _L // _U, hist_step, 0)

        def merged(i):
            hvec = histp[pl.ds(i * _L, _L)]
            for u in range(1, _U):
                hvec = hvec + histp[pl.ds(u * _NBINS + i * _L, _L)]
            return hvec

        lane = lax.iota(jnp.int32, _L)

        def find_desc(nvregs, load_vec, quota):
            def step(j, carry):
                above, found, bsel, absel = carry
                i = nvregs - 1 - j
                hvec = load_vec(i)
                sfx = lax.rev(plsc.cumsum(lax.rev(hvec, (0,))), (0,))
                incl = sfx + above
                strict = incl - hvec
                npass = jnp.sum((incl >= quota).astype(jnp.int32))
                asel = jnp.sum(jnp.where(lane == npass - 1, strict, 0))
                take = jnp.logical_and(jnp.logical_not(found), npass > 0)
                bsel = jnp.where(take, i * _L + npass - 1, bsel)
                absel = jnp.where(take, asel, absel)
                found = jnp.logical_or(found, npass > 0)
                return above + jnp.sum(hvec), found, bsel, absel

            _, _, bsel, absel = lax.fori_loop(
                0, nvregs, step,
                (jnp.int32(0), jnp.bool_(False), jnp.int32(0),
                 jnp.int32(0)))
            return bsel, absel

        bstar, above12 = find_desc(_NBINS // _L, merged, _K)
        quota2 = _K - above12

        def zero_p2(g, _):
            for u in range(_U):
                histp2[pl.ds((g * _U + u) * _L, _L)] = zeros
            return 0

        def hist2_step(g, _):
            for u in range(_U):
                i = g * _U + u
                bits = lax.bitcast_convert_type(row_v[pl.ds(i * _L, _L)],
                                                jnp.int32)
                inbin = (bits >> 20) == bstar
                d2 = (bits >> 8) & 0xFFF
                cnt, last = plsc.scan_count(d2, mask=inbin)
                plsc.addupdate_scatter(histp2, [u * 4096 + d2], cnt,
                                       mask=jnp.logical_and(last, inbin))
            return 0

        lax.fori_loop(0, 4096 // _L, zero_p2, 0)
        lax.fori_loop(0, _N // _L // _U, hist2_step, 0)

        def merged2(i):
            hvec = histp2[pl.ds(i * _L, _L)]
            for u in range(1, _U):
                hvec = hvec + histp2[pl.ds(u * 4096 + i * _L, _L)]
            return hvec

        b2star, above2 = find_desc(4096 // _L, merged2, quota2)
        quota3 = quota2 - above2
        pref24 = (bstar << 12) | b2star

        def zero_p3(g, _):
            for u in range(_U):
                hist_v[pl.ds((g * _U + u) * _L, _L)] = zeros
            return 0

        def hist3_step(g, _):
            for u in range(_U):
                i = g * _U + u
                bits = lax.bitcast_convert_type(row_v[pl.ds(i * _L, _L)],
                                                jnp.int32)
                inbin = (bits >> 8) == pref24
                d3 = bits & 0xFF
                cnt, last = plsc.scan_count(d3, mask=inbin)
                plsc.addupdate_scatter(hist_v, [u * 256 + d3], cnt,
                                       mask=jnp.logical_and(last, inbin))
            return 0

        def merged3(i):
            hvec = hist_v[pl.ds(i * _L, _L)]
            for u in range(1, _U):
                hvec = hvec + hist_v[pl.ds(u * 256 + i * _L, _L)]
            return hvec

        lax.fori_loop(0, _NBINS // _L // _U, zero_p3, 0)
        lax.fori_loop(0, _N // _L // _U, hist3_step, 0)
        b3star, _ = find_desc(256 // _L, merged3, quota3)
        thresh = (pref24 << 8) | b3star

        def pad_step(i, _):
            cka[pl.ds(i * _L, _L)] = jnp.full((_L,), _PAD_KEY, jnp.int32)
            return 0

        lax.fori_loop(0, _CAND // _L, pad_step, 0)

        def compact_step(g, off):
            def do_store(off):
                keys, masks, pops = [], [], []
                for u in range(_U):
                    i = g * _U + u
                    bits = lax.bitcast_convert_type(
                        row_v[pl.ds(i * _L, _L)], jnp.int32)
                    m = bits >= thresh
                    keys.append(_PAD_KEY - bits)
                    masks.append(m)
                    pops.append(jnp.sum(m.astype(jnp.int32)))
                for u in range(_U):
                    i = g * _U + u
                    plsc.store_compressed(cka.at[pl.ds(off, _L)],
                                          keys[u], mask=masks[u])
                    plsc.store_compressed(cia.at[pl.ds(off, _L)],
                                          i * _L + lane, mask=masks[u])
                    off = off + pops[u]
                return off

            return lax.cond(off <= _CAND - _U * _L, do_store,
                            lambda o: o, off)

        ncand = lax.fori_loop(0, _N // _L // _U, compact_step, jnp.int32(0))
        ngrp = (ncand + _U * _L - 1) // (_U * _L)

        for shift, (sk, si, dk, di) in zip(
                (0, 10, 20), ((cka, cia, ckb, cib),
                              (ckb, cib, cka, cia),
                              (cka, cia, ckb, cib))):
            lax.fori_loop(0, _U * _NBINS // _L // _U, zero_partials, 0)

            def dig_hist(g, _, sk=sk, shift=shift):
                for u in range(_U):
                    i = g * _U + u
                    d = (sk[pl.ds(i * _L, _L)] >> shift) & (_NBINS - 1)
                    cnt, last = plsc.scan_count(d)
                    plsc.addupdate_scatter(histp, [u * _NBINS + d], cnt,
                                           mask=last)
                return 0

            lax.fori_loop(0, ngrp, dig_hist, 0)

            def excl_step(i, carry):
                hvec = merged(i)
                hist_v[pl.ds(i * _L, _L)] = (plsc.cumsum(hvec) - hvec
                                             + carry)
                return carry + jnp.sum(hvec)

            lax.fori_loop(0, _NBINS // _L, excl_step, jnp.int32(0))

            def permute(g, _, sk=sk, si=si, dk=dk, di=di, shift=shift):
                for u in range(_U):
                    i = g * _U + u
                    k = sk[pl.ds(i * _L, _L)]
                    v = si[pl.ds(i * _L, _L)]
                    d = (k >> shift) & (_NBINS - 1)
                    cnt, last = plsc.scan_count(d)
                    pos = plsc.load_gather(hist_v, [d]) + cnt - 1
                    plsc.store_scatter(dk, [pos], k)
                    plsc.store_scatter(di, [pos], v)
                    plsc.addupdate_scatter(hist_v, [d], cnt, mask=last)
                return 0

            lax.fori_loop(0, ngrp, permute, 0)

        pltpu.sync_copy(cib.at[pl.ds(0, _K)], inds_hbm.at[row])


def _sc_topk(imp):
    mesh = plsc.VectorSubcoreMesh(core_axis_name="c", subcore_axis_name="s",
                                  num_cores=2, num_subcores=16)
    return pl.kernel(
        _sc_topk_body,
        out_type=jax.ShapeDtypeStruct((_B, _K), jnp.int32),
        mesh=mesh,
        compiler_params=pltpu.CompilerParams(needs_layout_passes=False),
        scratch_types=[
            pltpu.VMEM((_N,), jnp.float32),
            pltpu.VMEM((_U * _NBINS,), jnp.int32),
            pltpu.VMEM((_U * 4096,), jnp.int32),
            pltpu.VMEM((_NBINS,), jnp.int32),
            pltpu.VMEM((_CAND,), jnp.int32),
            pltpu.VMEM((_CAND,), jnp.int32),
            pltpu.VMEM((_CAND,), jnp.int32),
            pltpu.VMEM((_CAND,), jnp.int32),
        ],
    )(imp)


def _fc_body(hs_ref, Wfc_ref, bfc_ref, out_ref):
    hs = hs_ref[0]
    y = jax.lax.dot_general(
        hs, Wfc_ref[...],
        dimension_numbers=(((0,), (1,)), ((), ())),
        preferred_element_type=jnp.float32,
    )
    out_ref[...] = (y + bfc_ref[...])[None]


def _fc(h_sub, Wfc, bfc):
    return pl.pallas_call(
        _fc_body,
        grid=(_B,),
        in_specs=[
            pl.BlockSpec((1, _C, _K), lambda b: (b, 0, 0)),
            pl.BlockSpec((_OUT, _C), lambda b: (0, 0)),
            pl.BlockSpec((1, _OUT), lambda b: (0, 0)),
        ],
        out_specs=pl.BlockSpec((1, _K, _OUT), lambda b: (b, 0, 0)),
        out_shape=jax.ShapeDtypeStruct((_B, _K, _OUT), jnp.float32),
    )(h_sub, Wfc, bfc.reshape(1, _OUT))


def kernel(h, W1, b1, W2, b2, Wa, ba, Wfc, bfc):
    importance = _importance(h, W1, b1, W2, b2, Wa, ba)
    inds = _sc_topk(importance)
    h_sub = jnp.take_along_axis(h, inds[:, None, :], axis=2,
                                mode="clip")
    x = _fc(h_sub, Wfc, bfc)
    x = x.reshape(_B, _K, 3, 6, 8)
    return (x, inds, importance)


# speedup vs baseline: 1.2529x; 1.2529x over previous
import functools

import jax
import jax.numpy as jnp
from jax import lax
from jax.experimental import pallas as pl
from jax.experimental.pallas import tpu as pltpu
from jax.experimental.pallas import tpu_sc as plsc

_B, _C, _N = 8, 128, 65536
_K = 2048
_H1, _H2 = 64, 64
_OUT = 144
_NT = 4096
_L = 16
_NBINS = 1024
_CAND = 4096
_PAD_KEY = 0x3FFFFFFF


def _imp_body(h_ref, W1_ref, b1_ref, W2_ref, b2_ref, Wa_ref, ba_ref, out_ref):
    hb = h_ref[0]
    f1 = jax.lax.dot(W1_ref[...], hb, preferred_element_type=jnp.float32)
    f1 = f1 + b1_ref[...].T
    f1 = jnp.where(f1 >= 0, f1, 0.1 * f1)
    f2 = jax.lax.dot(W2_ref[...], f1, preferred_element_type=jnp.float32)
    f2 = f2 + b2_ref[...].T
    f2 = jnp.where(f2 >= 0, f2, 0.1 * f2)
    a = jax.lax.dot(Wa_ref[...], f2, preferred_element_type=jnp.float32)
    a = a + ba_ref[...]
    out_ref[...] = jnp.clip(jax.nn.sigmoid(a), 0.0, 1.0)[None]


def _importance(h, W1, b1, W2, b2, Wa, ba):
    grid = (_B, _N // _NT)
    return pl.pallas_call(
        _imp_body,
        grid=grid,
        in_specs=[
            pl.BlockSpec((1, _C, _NT), lambda b, n: (b, 0, n)),
            pl.BlockSpec((_H1, _C), lambda b, n: (0, 0)),
            pl.BlockSpec((1, _H1), lambda b, n: (0, 0)),
            pl.BlockSpec((_H2, _H1), lambda b, n: (0, 0)),
            pl.BlockSpec((1, _H2), lambda b, n: (0, 0)),
            pl.BlockSpec((1, _H2), lambda b, n: (0, 0)),
            pl.BlockSpec((1, 1), lambda b, n: (0, 0)),
        ],
        out_specs=pl.BlockSpec((1, 1, _NT), lambda b, n: (b, 0, n)),
        out_shape=jax.ShapeDtypeStruct((_B, 1, _N), jnp.float32),
    )(h, W1, b1.reshape(1, _H1), W2, b2.reshape(1, _H2), Wa,
      ba.reshape(1, 1)).reshape(_B, _N)


_U = 4


def _sc_topk_body(imp_hbm, inds_hbm, row_v, histp, histp2, hist_v, cka,
                  cia, ckb, cib):
    cid = lax.axis_index("c")
    sid = lax.axis_index("s")
    row = cid * (_B // 2) + sid

    @pl.when(sid < _B // 2)
    def _():
        pltpu.sync_copy(imp_hbm.at[row], row_v)
        zeros = jnp.zeros((_L,), jnp.int32)

        def zero_partials(g, _):
            for u in range(_U):
                histp[pl.ds((g * _U + u) * _L, _L)] = zeros
            return 0

        def hist_step(g, _):
            for u in range(_U):
                i = g * _U + u
                bits = lax.bitcast_convert_type(row_v[pl.ds(i * _L, _L)],
                                                jnp.int32)
                b = jnp.minimum(bits >> 20, _NBINS - 1)
                cnt, last = plsc.scan_count(b)
                plsc.addupdate_scatter(histp, [u * _NBINS + b], cnt,
                                       mask=last)
            return 0

        lax.fori_loop(0, _U * _NBINS // _L // _U, zero_partials, 0)
        lax.fori_loop(0, _N // _L // _U, hist_step, 0)

        def merged(i):
            hvec = histp[pl.ds(i * _L, _L)]
            for u in range(1, _U):
                hvec = hvec + histp[pl.ds(u * _NBINS + i * _L, _L)]
            return hvec

        lane = lax.iota(jnp.int32, _L)

        def find_desc(nvregs, load_vec, quota):
            def step(j, carry):
                above, found, bsel, absel, csel = carry
                i = nvregs - 1 - j
                hvec = load_vec(i)
                sfx = lax.rev(plsc.cumsum(lax.rev(hvec, (0,))), (0,))
                incl = sfx + above
                strict = incl - hvec
                npass = jnp.sum((incl >= quota).astype(jnp.int32))
                asel = jnp.sum(jnp.where(lane == npass - 1, strict, 0))
                hsel = jnp.sum(jnp.where(lane == npass - 1, hvec, 0))
                take = jnp.logical_and(jnp.logical_not(found), npass > 0)
                bsel = jnp.where(take, i * _L + npass - 1, bsel)
                absel = jnp.where(take, asel, absel)
                csel = jnp.where(take, hsel, csel)
                found = jnp.logical_or(found, npass > 0)
                return above + jnp.sum(hvec), found, bsel, absel, csel

            _, _, bsel, absel, csel = lax.fori_loop(
                0, nvregs, step,
                (jnp.int32(0), jnp.bool_(False), jnp.int32(0),
                 jnp.int32(0), jnp.int32(0)))
            return bsel, absel, csel

        bstar, above12, count12 = find_desc(_NBINS // _L, merged, _K)
        quota2 = _K - above12

        def zero_p2(g, _):
            for u in range(_U):
                histp2[pl.ds((g * _U + u) * _L, _L)] = zeros
            return 0

        def hist2_step(g, _):
            for u in range(_U):
                i = g * _U + u
                bits = lax.bitcast_convert_type(row_v[pl.ds(i * _L, _L)],
                                                jnp.int32)
                inbin = (bits >> 20) == bstar
                d2 = (bits >> 8) & 0xFFF
                cnt, last = plsc.scan_count(d2, mask=inbin)
                plsc.addupdate_scatter(histp2, [u * 4096 + d2], cnt,
                                       mask=jnp.logical_and(last, inbin))
            return 0

        def merged2(i):
            hvec = histp2[pl.ds(i * _L, _L)]
            for u in range(1, _U):
                hvec = hvec + histp2[pl.ds(u * 4096 + i * _L, _L)]
            return hvec

        def merged3(i):
            hvec = hist_v[pl.ds(i * _L, _L)]
            for u in range(1, _U):
                hvec = hvec + hist_v[pl.ds(u * 256 + i * _L, _L)]
            return hvec

        def zero_p3(g, _):
            for u in range(_U):
                hist_v[pl.ds((g * _U + u) * _L, _L)] = zeros
            return 0

        def refine(_):
            lax.fori_loop(0, 4096 // _L, zero_p2, 0)
            lax.fori_loop(0, _N // _L // _U, hist2_step, 0)
            b2star, above2, _c2 = find_desc(4096 // _L, merged2, quota2)
            quota3 = quota2 - above2
            pref24 = (bstar << 12) | b2star

            def hist3_step(g, _):
                for u in range(_U):
                    i = g * _U + u
                    bits = lax.bitcast_convert_type(
                        row_v[pl.ds(i * _L, _L)], jnp.int32)
                    inbin = (bits >> 8) == pref24
                    d3 = bits & 0xFF
                    cnt, last = plsc.scan_count(d3, mask=inbin)
                    plsc.addupdate_scatter(
                        hist_v, [u * 256 + d3], cnt,
                        mask=jnp.logical_and(last, inbin))
                return 0

            lax.fori_loop(0, _NBINS // _L // _U, zero_p3, 0)
            lax.fori_loop(0, _N // _L // _U, hist3_step, 0)
            b3star, _a3, _c3 = find_desc(256 // _L, merged3, quota3)
            return (pref24 << 8) | b3star

        thresh = lax.cond(above12 + count12 > _CAND - _U * _L,
                          refine, lambda _: bstar << 20, 0)

        def pad_step(i, _):
            cka[pl.ds(i * _L, _L)] = jnp.full((_L,), _PAD_KEY, jnp.int32)
            return 0

        lax.fori_loop(0, _CAND // _L, pad_step, 0)

        def compact_step(g, off):
            def do_store(off):
                keys, masks, pops = [], [], []
                for u in range(_U):
                    i = g * _U + u
                    bits = lax.bitcast_convert_type(
                        row_v[pl.ds(i * _L, _L)], jnp.int32)
                    m = bits >= thresh
                    keys.append(_PAD_KEY - bits)
                    masks.append(m)
                    pops.append(jnp.sum(m.astype(jnp.int32)))
                for u in range(_U):
                    i = g * _U + u
                    plsc.store_compressed(cka.at[pl.ds(off, _L)],
                                          keys[u], mask=masks[u])
                    plsc.store_compressed(cia.at[pl.ds(off, _L)],
                                          i * _L + lane, mask=masks[u])
                    off = off + pops[u]
                return off

            return lax.cond(off <= _CAND - _U * _L, do_store,
                            lambda o: o, off)

        ncand = lax.fori_loop(0, _N // _L // _U, compact_step, jnp.int32(0))
        ngrp = (ncand + _U * _L - 1) // (_U * _L)

        for shift, (sk, si, dk, di) in zip(
                (0, 10, 20), ((cka, cia, ckb, cib),
                              (ckb, cib, cka, cia),
                              (cka, cia, ckb, cib))):
            lax.fori_loop(0, _U * _NBINS // _L // _U, zero_partials, 0)

            def dig_hist(g, _, sk=sk, shift=shift):
                for u in range(_U):
                    i = g * _U + u
                    d = (sk[pl.ds(i * _L, _L)] >> shift) & (_NBINS - 1)
                    cnt, last = plsc.scan_count(d)
                    plsc.addupdate_scatter(histp, [u * _NBINS + d], cnt,
                                           mask=last)
                return 0

            lax.fori_loop(0, ngrp, dig_hist, 0)

            def excl_step(i, carry):
                hvec = merged(i)
                hist_v[pl.ds(i * _L, _L)] = (plsc.cumsum(hvec) - hvec
                                             + carry)
                return carry + jnp.sum(hvec)

            lax.fori_loop(0, _NBINS // _L, excl_step, jnp.int32(0))

            def permute(g, _, sk=sk, si=si, dk=dk, di=di, shift=shift):
                for u in range(_U):
                    i = g * _U + u
                    k = sk[pl.ds(i * _L, _L)]
                    v = si[pl.ds(i * _L, _L)]
                    d = (k >> shift) & (_NBINS - 1)
                    cnt, last = plsc.scan_count(d)
                    pos = plsc.load_gather(hist_v, [d]) + cnt - 1
                    plsc.store_scatter(dk, [pos], k)
                    plsc.store_scatter(di, [pos], v)
                    plsc.addupdate_scatter(hist_v, [d], cnt, mask=last)
                return 0

            lax.fori_loop(0, ngrp, permute, 0)

        pltpu.sync_copy(cib.at[pl.ds(0, _K)], inds_hbm.at[row])


def _sc_topk(imp):
    mesh = plsc.VectorSubcoreMesh(core_axis_name="c", subcore_axis_name="s",
                                  num_cores=2, num_subcores=16)
    return pl.kernel(
        _sc_topk_body,
        out_type=jax.ShapeDtypeStruct((_B, _K), jnp.int32),
        mesh=mesh,
        compiler_params=pltpu.CompilerParams(needs_layout_passes=False),
        scratch_types=[
            pltpu.VMEM((_N,), jnp.float32),
            pltpu.VMEM((_U * _NBINS,), jnp.int32),
            pltpu.VMEM((_U * 4096,), jnp.int32),
            pltpu.VMEM((_NBINS,), jnp.int32),
            pltpu.VMEM((_CAND,), jnp.int32),
            pltpu.VMEM((_CAND,), jnp.int32),
            pltpu.VMEM((_CAND,), jnp.int32),
            pltpu.VMEM((_CAND,), jnp.int32),
        ],
    )(imp)


def _fc_body(hs_ref, Wfc_ref, bfc_ref, out_ref):
    hs = hs_ref[0]
    y = jax.lax.dot_general(
        hs, Wfc_ref[...],
        dimension_numbers=(((0,), (1,)), ((), ())),
        preferred_element_type=jnp.float32,
    )
    out_ref[...] = (y + bfc_ref[...])[None]


def _fc(h_sub, Wfc, bfc):
    return pl.pallas_call(
        _fc_body,
        grid=(_B,),
        in_specs=[
            pl.BlockSpec((1, _C, _K), lambda b: (b, 0, 0)),
            pl.BlockSpec((_OUT, _C), lambda b: (0, 0)),
            pl.BlockSpec((1, _OUT), lambda b: (0, 0)),
        ],
        out_specs=pl.BlockSpec((1, _K, _OUT), lambda b: (b, 0, 0)),
        out_shape=jax.ShapeDtypeStruct((_B, _K, _OUT), jnp.float32),
    )(h_sub, Wfc, bfc.reshape(1, _OUT))


def kernel(h, W1, b1, W2, b2, Wa, ba, Wfc, bfc):
    importance = _importance(h, W1, b1, W2, b2, Wa, ba)
    inds = _sc_topk(importance)
    h_sub = jnp.take_along_axis(h, inds[:, None, :], axis=2,
                                mode="clip")
    x = _fc(h_sub, Wfc, bfc)
    x = x.reshape(_B, _K, 3, 6, 8)
    return (x, inds, importance)
